# Initial kernel scaffold; baseline (speedup 1.0000x reference)
import jax
import jax.numpy as jnp
from jax.experimental import pallas as pl


def _tc_body(aa_ref, crg_ref, res_ref, atom_ref, crgt_ref, wf_ref, bf_ref, out_ref):
    B = aa_ref.shape[1]
    t_res = jnp.dot(res_ref[...], wf_ref[0:64, :], preferred_element_type=jnp.float32)
    t_atom = jnp.dot(atom_ref[...], wf_ref[64:128, :], preferred_element_type=jnp.float32)
    t_crg = jnp.dot(crgt_ref[...], wf_ref[128:144, :], preferred_element_type=jnp.float32)
    aa_b = aa_ref[0, :]
    crg_b = crg_ref[0, :]
    oh_aa = (jax.lax.broadcasted_iota(jnp.int32, (B, 21), 1) == aa_b[:, None]).astype(jnp.float32)
    oh_cr = (jax.lax.broadcasted_iota(jnp.int32, (B, 3), 1) == crg_b[:, None]).astype(jnp.float32)
    rows = (jnp.dot(oh_aa, t_res, preferred_element_type=jnp.float32)
            + jnp.dot(oh_cr, t_crg, preferred_element_type=jnp.float32)
            + bf_ref[...][None, :])
    out_ref[...] = rows[:, None, :] + t_atom[None, :, :]


def kernel(aa, pos14, atom_mask, phys, crg, res_table, atom_table, crg_table, Wp, bp, Wf, bf):
    N, L = aa.shape
    NL = N * L
    B = 512
    grid = NL // B

    aa_f = aa.reshape(1, NL)
    crg_f = crg.reshape(1, NL)

    full = lambda shape: pl.BlockSpec(shape, lambda i: (0,) * len(shape))
    feats = pl.pallas_call(
        _tc_body,
        grid=(grid,),
        in_specs=[
            pl.BlockSpec((1, B), lambda i: (0, i)),
            pl.BlockSpec((1, B), lambda i: (0, i)),
            full((21, 64)),
            full((14, 64)),
            full((3, 16)),
            full((144, 128)),
            full((128,)),
        ],
        out_specs=pl.BlockSpec((B, 14, 128), lambda i: (i, 0, 0)),
        out_shape=jax.ShapeDtypeStruct((NL, 14, 128), jnp.float32),
    )(aa_f, crg_f, res_table, atom_table, crg_table, Wf, bf)

    feats = feats.reshape(N, L * 14, 128)
    coors = pos14.reshape(N, L * 14, 3)
    mask = atom_mask.reshape(N, L * 14)
    return (feats, coors, mask)


# speedup vs baseline: 9.0978x; 9.0978x over previous
import functools

import jax
import jax.numpy as jnp
from jax import lax
from jax.experimental import pallas as pl
from jax.experimental.pallas import tpu as pltpu
from jax.experimental.pallas import tpu_sc as plsc

_W = 32
_D = 14 * 128


def _table_body(res_ref, atom_ref, crgt_ref, wf_ref, bf_ref, out_ref):
    t_res = jnp.dot(res_ref[...], wf_ref[0:64, :], preferred_element_type=jnp.float32)
    t_atom = jnp.dot(atom_ref[...], wf_ref[64:128, :], preferred_element_type=jnp.float32)
    t_crg = jnp.dot(crgt_ref[...], wf_ref[128:144, :], preferred_element_type=jnp.float32)
    row_r = jax.lax.broadcasted_iota(jnp.int32, (63, 21), 0) // 3
    col_r = jax.lax.broadcasted_iota(jnp.int32, (63, 21), 1)
    e_res = (col_r == row_r).astype(jnp.float32)
    row_c = jax.lax.broadcasted_iota(jnp.int32, (63, 3), 0) % 3
    col_c = jax.lax.broadcasted_iota(jnp.int32, (63, 3), 1)
    e_crg = (col_c == row_c).astype(jnp.float32)
    rows63 = (jnp.dot(e_res, t_res, preferred_element_type=jnp.float32)
              + jnp.dot(e_crg, t_crg, preferred_element_type=jnp.float32)
              + bf_ref[...][None, :])
    for a in range(14):
        out_ref[:, a * 128:(a + 1) * 128] = rows63 + t_atom[a:a + 1, :]


def _build_t3(res_table, atom_table, crg_table, Wf, bf):
    full = lambda shape: pl.BlockSpec(shape, lambda: (0,) * len(shape))
    return pl.pallas_call(
        _table_body,
        in_specs=[full((21, 64)), full((14, 64)), full((3, 16)),
                  full((144, 128)), full((128,))],
        out_specs=full((63, _D)),
        out_shape=jax.ShapeDtypeStruct((63, _D), jnp.float32),
    )(res_table, atom_table, crg_table, Wf, bf)


def _sc_gather(t3, aa2, crg2, nl):
    mesh = plsc.VectorSubcoreMesh(core_axis_name="c", subcore_axis_name="s")

    @functools.partial(
        pl.kernel, mesh=mesh,
        out_type=jax.ShapeDtypeStruct((nl, _D), jnp.float32),
        scratch_types=[pltpu.VMEM((_W,), jnp.int32)],
    )
    def sc_kernel(t3_hbm, aa_hbm, crg_hbm, out_hbm, idx_v):
        def body(aa_vmem, crg_vmem, o_vmem):
            @pl.loop(0, _W // 16)
            def _(i):
                s = pl.ds(i * 16, 16)
                idx_v[s] = aa_vmem[s] * 3 + crg_vmem[s]

            pltpu.sync_copy(t3_hbm.at[idx_v], o_vmem)

        pltpu.emit_pipeline(
            body,
            grid=(nl // _W,),
            in_specs=[pl.BlockSpec((_W,), lambda i: (i,)),
                      pl.BlockSpec((_W,), lambda i: (i,))],
            out_specs=[pl.BlockSpec((_W, _D), lambda i: (i, 0))],
            core_axis_name=("c", "s"),
            dimension_semantics=(pltpu.PARALLEL,),
        )(aa_hbm, crg_hbm, out_hbm)

    return sc_kernel(t3, aa2, crg2)


def kernel(aa, pos14, atom_mask, phys, crg, res_table, atom_table, crg_table, Wp, bp, Wf, bf):
    N, L = aa.shape
    NL = N * L
    t3 = _build_t3(res_table, atom_table, crg_table, Wf, bf)
    out = _sc_gather(t3, aa.reshape(1, NL), crg.reshape(1, NL), NL)
    feats = out.reshape(N, L * 14, 128)
    coors = pos14.reshape(N, L * 14, 3)
    mask = atom_mask.reshape(N, L * 14)
    return (feats, coors, mask)
